# baseline (speedup 1.0000x reference)
import functools

import jax
import jax.numpy as jnp
from jax import lax
from jax.experimental import pallas as pl
from jax.experimental.pallas import tpu as pltpu
from jax.experimental.pallas import tpu_sc as plsc

B = 1024
D = 512
M = 100000
K = 50
C = 10

L = 16
CW = 128
BB = 256
BM = 2048
NJ = 49
MP = NJ * BM
CPS = BM // CW
NCHUNK = MP // CW
CMOUT = 896
PAD_CHUNK = NCHUNK - 1
CAP = 64
K1_BISECT_ITERS = 22
K3_BISECT_ITERS = 38

NC = 2
NS = 16
NW = NC * NS
ROWS_PER_W = B // NW
NPAIR = ROWS_PER_W // 2


def _k1_body(x_ref, k_ref, v_ref, s_ref, cm_ref, tau_ref,
             xn_ref, cmt_ref, cmw_ref):
    j = pl.program_id(1)

    @pl.when(j == 0)
    def _():
        xv = x_ref[...]
        n = jnp.sqrt(jnp.sum(xv * xv, axis=1, keepdims=True))
        xn_ref[...] = xv / jnp.clip(n, 1e-12, None)

    s = lax.dot_general(xn_ref[...], k_ref[...], (((1,), (1,)), ((), ())),
                        preferred_element_type=jnp.float32)
    si = lax.bitcast_convert_type(s, jnp.int32)
    s = lax.bitcast_convert_type((si & ~15) | v_ref[...], jnp.float32)

    @pl.when(j == NJ - 1)
    def _():
        col = lax.broadcasted_iota(jnp.int32, (1, BM), 1) + j * BM
        sm = jnp.where(col < M, s, -2.0)
        s_ref[...] = sm.reshape(BB, CPS, CW)

    @pl.when(j < NJ - 1)
    def _():
        s_ref[...] = s.reshape(BB, CPS, CW)

    cm = jnp.max(s_ref[...], axis=2)
    cmt_ref[j] = cm

    grp = lax.broadcasted_iota(jnp.int32, (1, 128), 1) // CPS
    old = jnp.where(j % 8 == 0,
                    jnp.full((BB, 128), -2.0, jnp.float32), cmw_ref[...])
    cmw_ref[...] = jnp.where(grp == j % 8, jnp.tile(cm, (1, 8)), old)

    @pl.when((j % 8 == 7) | (j == NJ - 1))
    def _():
        cm_ref[...] = cmw_ref[...]

    @pl.when(j == NJ - 1)
    def _():
        cmall = cmt_ref[...]

        def it(_, c):
            lo, hi = c
            mid = (lo + hi) * 0.5
            cnt = jnp.sum(jnp.sum(
                (cmall >= mid[:, :, None]).astype(jnp.float32), axis=2),
                axis=0, keepdims=True)
            p = cnt >= K
            return jnp.where(p, mid, lo), jnp.where(p, hi, mid)

        lo = jnp.full((1, BB), -1.001, jnp.float32)
        hi = jnp.full((1, BB), 1.001, jnp.float32)
        lo, hi = lax.fori_loop(0, K1_BISECT_ITERS, it, (lo, hi))
        tau_ref[...] = lo


def _k1(x, keys, vals_pad):
    return pl.pallas_call(
        _k1_body,
        grid=(B // BB, NJ),
        in_specs=[
            pl.BlockSpec((BB, D), lambda i, j: (i, 0)),
            pl.BlockSpec((BM, D), lambda i, j: (j, 0)),
            pl.BlockSpec((1, BM), lambda i, j: (0, j)),
        ],
        out_specs=[
            pl.BlockSpec((BB, CPS, CW), lambda i, j: (i, j, 0)),
            pl.BlockSpec((BB, 128), lambda i, j: (i, j // 8)),
            pl.BlockSpec((1, BB), lambda i, j: (0, i)),
        ],
        out_shape=[
            jax.ShapeDtypeStruct((B, NCHUNK, CW), jnp.float32),
            jax.ShapeDtypeStruct((B, CMOUT), jnp.float32),
            jax.ShapeDtypeStruct((1, B), jnp.float32),
        ],
        scratch_shapes=[
            pltpu.VMEM((BB, D), jnp.float32),
            pltpu.VMEM((NJ, BB, CPS), jnp.float32),
            pltpu.VMEM((BB, 128), jnp.float32),
        ],
    )(x, keys, vals_pad)


def _k2_body(cm2_hbm, tau_hbm, stbl_hbm, out_s_hbm,
             cm_a, cm_b, ids_a, ids_b, gids_a, gids_b, sbuf_a, sbuf_b, tau_v,
             semc_a, semc_b, semg_a, semg_b, semw_a, semw_b):
    wid = lax.axis_index("s") * NC + lax.axis_index("c")
    base = wid * ROWS_PER_W
    pltpu.sync_copy(tau_hbm.at[pl.ds(base * 1, ROWS_PER_W)],
                    tau_v.at[pl.ds(0, ROWS_PER_W)])
    lane = lax.iota(jnp.int32, L)

    def cm_win(octet):
        return cm2_hbm.at[pl.ds(base + 8 * octet, 8), :]

    def fire_cm(octet, cm_ref, sem):
        return pltpu.async_copy(cm_win(octet), cm_ref, sem)

    def drain_cm(cm_ref, sem):
        pltpu.make_async_copy(cm_win(0), cm_ref, sem).wait()

    def scan_pair(pair, cm_ref, ids_ref, gids_ref):
        r0 = base + 2 * pair
        p2 = pair % 4

        def one_row(rr, tau_vec):
            off = CAP * rr

            def body(jj, pos):
                v = cm_ref[2 * p2 + rr, pl.ds(jj * L, L)]
                msk = v >= tau_vec
                ids = lane + jj * L
                mi = jnp.where(msk, 1, 0)
                slots = off + pos + plsc.cumsum(mi) - 1
                plsc.store_scatter(ids_ref, [slots], ids,
                                   mask=msk & (slots < off + CAP))
                return jnp.minimum(pos + jnp.sum(mi), CAP)

            npos = lax.fori_loop(0, NCHUNK // L, body, jnp.int32(0))
            for k in range(CAP // L):
                sl = ids_ref[pl.ds(off + k * L, L)]
                slot = lane + k * L
                sl = jnp.where(slot >= npos, PAD_CHUNK, sl)
                ids_ref[pl.ds(off + k * L, L)] = sl
                gids_ref[pl.ds(off + k * L, L)] = sl + (r0 + rr) * NCHUNK

        i0 = 2 * pair
        one_row(0, jnp.full((L,), tau_v[pl.ds(i0, L)][0], jnp.float32))
        one_row(1, jnp.full((L,), tau_v[pl.ds(i0 + 1, L)][0], jnp.float32))

    def fire_gather(pair, gids_ref, sbuf, sem):
        pltpu.async_copy(stbl_hbm.at[gids_ref], sbuf, sem)

    def drain_gather(sbuf, sem):
        pltpu.make_async_copy(stbl_hbm.at[pl.ds(0, 2 * CAP)], sbuf, sem).wait()

    def out_win(pair):
        return out_s_hbm.at[pl.ds((base + 2 * pair) * CAP, 2 * CAP)]

    def fire_write(pair, sbuf, sem):
        pltpu.async_copy(sbuf, out_win(pair), sem)

    def drain_write(pair, sbuf, sem):
        pltpu.make_async_copy(sbuf, out_win(pair), sem).wait()

    cmbufs = ((cm_a, semc_a), (cm_b, semc_b))
    bufs = ((ids_a, gids_a, sbuf_a, semg_a, semw_a),
            (ids_b, gids_b, sbuf_b, semg_b, semw_b))
    NOCT = NPAIR // 4

    fire_cm(0, cm_a, semc_a)
    if NOCT > 1:
        fire_cm(1, cm_b, semc_b)
    for o in range(NOCT):
        cm, semc = cmbufs[o % 2]
        drain_cm(cm, semc)
        for p2 in range(4):
            g = 4 * o + p2
            ids, gids, sbuf, semg, semw = bufs[g % 2]
            scan_pair(g, cm, ids, gids)
            if g >= 2:
                drain_write(g - 2, sbuf, semw)
            fire_gather(g, gids, sbuf, semg)
            if g >= 1:
                _, _, psb, psemg, psemw = bufs[(g - 1) % 2]
                drain_gather(psb, psemg)
                fire_write(g - 1, psb, psemw)
        if o + 2 < NOCT:
            fire_cm(o + 2, cm, semc)
    _, _, lsb, lsemg, lsemw = bufs[(NPAIR - 1) % 2]
    drain_gather(lsb, lsemg)
    fire_write(NPAIR - 1, lsb, lsemw)
    drain_write(NPAIR - 2, bufs[(NPAIR - 2) % 2][2], bufs[(NPAIR - 2) % 2][4])
    drain_write(NPAIR - 1, lsb, lsemw)


@functools.lru_cache(maxsize=1)
def _k2_sc():
    return pl.kernel(
        _k2_body,
        out_type=jax.ShapeDtypeStruct((B * CAP, CW), jnp.float32),
        mesh=plsc.VectorSubcoreMesh(core_axis_name="c", subcore_axis_name="s",
                                    num_cores=NC, num_subcores=NS),
        compiler_params=pltpu.CompilerParams(needs_layout_passes=False),
        scratch_types=[
            pltpu.VMEM((8, CMOUT), jnp.float32),
            pltpu.VMEM((8, CMOUT), jnp.float32),
            pltpu.VMEM((2 * CAP,), jnp.int32),
            pltpu.VMEM((2 * CAP,), jnp.int32),
            pltpu.VMEM((2 * CAP,), jnp.int32),
            pltpu.VMEM((2 * CAP,), jnp.int32),
            pltpu.VMEM((2 * CAP, CW), jnp.float32),
            pltpu.VMEM((2 * CAP, CW), jnp.float32),
            pltpu.VMEM((ROWS_PER_W + L,), jnp.float32),
            pltpu.SemaphoreType.DMA,
            pltpu.SemaphoreType.DMA,
            pltpu.SemaphoreType.DMA,
            pltpu.SemaphoreType.DMA,
            pltpu.SemaphoreType.DMA,
            pltpu.SemaphoreType.DMA,
        ],
    )


NCAND = CAP * CW


def _k3_body(s_ref, o_ref):
    s = s_ref[...]
    lbl = lax.bitcast_convert_type(s, jnp.int32) & 15

    def it(_, c):
        lo, hi = c
        mid = (lo + hi) * 0.5
        cnt = jnp.sum((s >= mid).astype(jnp.float32), axis=1, keepdims=True)
        p = cnt >= K
        return jnp.where(p, mid, lo), jnp.where(p, hi, mid)

    lo = jnp.full((BB, 1), -1.001, jnp.float32)
    hi = jnp.full((BB, 1), 1.001, jnp.float32)
    lo, hi = lax.fori_loop(0, K3_BISECT_ITERS, it, (lo, hi))
    ms = jnp.where(s >= lo, s, 0.0)
    cols = [jnp.sum(jnp.where(lbl == c, ms, 0.0), axis=1, keepdims=True)
            for c in range(C)]
    o_ref[...] = jnp.concatenate(cols, axis=1)


def _k3(cand_s):
    return pl.pallas_call(
        _k3_body,
        grid=(B // BB,),
        in_specs=[pl.BlockSpec((BB, NCAND), lambda i: (i, 0))],
        out_specs=pl.BlockSpec((BB, C), lambda i: (i, 0)),
        out_shape=jax.ShapeDtypeStruct((B, C), jnp.float32),
    )(cand_s)


def kernel(x, keys, values):
    vals_pad = jnp.pad(values, (0, MP - M)).reshape(1, MP)
    scores, cmax, tau = _k1(x, keys, vals_pad)
    cand_s = _k2_sc()(cmax, tau.reshape(-1),
                      scores.reshape(B * NCHUNK, CW))
    return _k3(cand_s.reshape(B, NCAND))


# speedup vs baseline: 1.0079x; 1.0079x over previous
import functools

import jax
import jax.numpy as jnp
from jax import lax
from jax.experimental import pallas as pl
from jax.experimental.pallas import tpu as pltpu
from jax.experimental.pallas import tpu_sc as plsc

B = 1024
D = 512
M = 100000
K = 50
C = 10

L = 16
CW = 128
BB = 256
BM = 2048
NJ = 49
MP = NJ * BM
CPS = BM // CW
NCHUNK = MP // CW
CMOUT = 896
PAD_CHUNK = NCHUNK - 1
CAP = 64
K1_BISECT_ITERS = 22
K3_BISECT_ITERS = 38

NC = 2
NS = 16
NW = NC * NS
ROWS_PER_W = B // NW
NPAIR = ROWS_PER_W // 2


def _k1_body(x_ref, k_ref, v_ref, s_ref, cm_ref, tau_ref,
             xn_ref, cmt_ref, cmw_ref):
    j = pl.program_id(1)

    @pl.when(j == 0)
    def _():
        xv = x_ref[...]
        n = jnp.sqrt(jnp.sum(xv * xv, axis=1, keepdims=True))
        xn_ref[...] = xv / jnp.clip(n, 1e-12, None)

    s = lax.dot_general(xn_ref[...], k_ref[...], (((1,), (1,)), ((), ())),
                        preferred_element_type=jnp.float32)
    si = lax.bitcast_convert_type(s, jnp.int32)
    s = lax.bitcast_convert_type((si & ~15) | v_ref[...], jnp.float32)

    col = lax.broadcasted_iota(jnp.int32, (1, BM), 1) + j * BM
    s = jnp.where(col < M, s, -2.0)
    s3 = s.reshape(BB, CPS, CW)
    s_ref[...] = s3
    cm = jnp.max(s3, axis=2)
    cmt_ref[j] = cm

    grp = lax.broadcasted_iota(jnp.int32, (1, 128), 1) // CPS
    old = jnp.where(j % 8 == 0,
                    jnp.full((BB, 128), -2.0, jnp.float32), cmw_ref[...])
    cmw_ref[...] = jnp.where(grp == j % 8, jnp.tile(cm, (1, 8)), old)

    @pl.when((j % 8 == 7) | (j == NJ - 1))
    def _():
        cm_ref[...] = cmw_ref[...]

    @pl.when(j == NJ - 1)
    def _():
        cmall = cmt_ref[...]

        def it(_, c):
            lo, hi = c
            mid = (lo + hi) * 0.5
            cnt = jnp.sum(jnp.sum(
                (cmall >= mid[:, :, None]).astype(jnp.float32), axis=2),
                axis=0, keepdims=True)
            p = cnt >= K
            return jnp.where(p, mid, lo), jnp.where(p, hi, mid)

        lo = jnp.full((1, BB), -1.001, jnp.float32)
        hi = jnp.full((1, BB), 1.001, jnp.float32)
        lo, hi = lax.fori_loop(0, K1_BISECT_ITERS, it, (lo, hi))
        tau_ref[...] = lo


def _k1(x, keys, vals_pad):
    return pl.pallas_call(
        _k1_body,
        grid=(B // BB, NJ),
        in_specs=[
            pl.BlockSpec((BB, D), lambda i, j: (i, 0)),
            pl.BlockSpec((BM, D), lambda i, j: (j, 0)),
            pl.BlockSpec((1, BM), lambda i, j: (0, j)),
        ],
        out_specs=[
            pl.BlockSpec((BB, CPS, CW), lambda i, j: (i, j, 0)),
            pl.BlockSpec((BB, 128), lambda i, j: (i, j // 8)),
            pl.BlockSpec((1, BB), lambda i, j: (0, i)),
        ],
        out_shape=[
            jax.ShapeDtypeStruct((B, NCHUNK, CW), jnp.float32),
            jax.ShapeDtypeStruct((B, CMOUT), jnp.float32),
            jax.ShapeDtypeStruct((1, B), jnp.float32),
        ],
        scratch_shapes=[
            pltpu.VMEM((BB, D), jnp.float32),
            pltpu.VMEM((NJ, BB, CPS), jnp.float32),
            pltpu.VMEM((BB, 128), jnp.float32),
        ],
    )(x, keys, vals_pad)


def _k2_body(cm2_hbm, tau_hbm, stbl_hbm, out_s_hbm,
             cm_a, cm_b, ids_a, ids_b, gids_a, gids_b, sbuf_a, sbuf_b, tau_v,
             semc_a, semc_b, semg_a, semg_b, semw_a, semw_b):
    wid = lax.axis_index("s") * NC + lax.axis_index("c")
    base = wid * ROWS_PER_W
    pltpu.sync_copy(tau_hbm.at[pl.ds(base * 1, ROWS_PER_W)],
                    tau_v.at[pl.ds(0, ROWS_PER_W)])
    lane = lax.iota(jnp.int32, L)

    def cm_win(octet):
        return cm2_hbm.at[pl.ds(base + 8 * octet, 8), :]

    def fire_cm(octet, cm_ref, sem):
        return pltpu.async_copy(cm_win(octet), cm_ref, sem)

    def drain_cm(cm_ref, sem):
        pltpu.make_async_copy(cm_win(0), cm_ref, sem).wait()

    def scan_pair(pair, cm_ref, ids_ref, gids_ref):
        r0 = base + 2 * pair
        p2 = pair % 4

        def one_row(rr, tau_vec):
            off = CAP * rr

            def body(jj, pos):
                v = cm_ref[2 * p2 + rr, pl.ds(jj * L, L)]
                msk = v >= tau_vec
                ids = lane + jj * L
                mi = jnp.where(msk, 1, 0)
                slots = off + pos + plsc.cumsum(mi) - 1
                plsc.store_scatter(ids_ref, [slots], ids,
                                   mask=msk & (slots < off + CAP))
                return jnp.minimum(pos + jnp.sum(mi), CAP)

            npos = lax.fori_loop(0, NCHUNK // L, body, jnp.int32(0))
            for k in range(CAP // L):
                sl = ids_ref[pl.ds(off + k * L, L)]
                slot = lane + k * L
                sl = jnp.where(slot >= npos, PAD_CHUNK, sl)
                ids_ref[pl.ds(off + k * L, L)] = sl
                gids_ref[pl.ds(off + k * L, L)] = sl + (r0 + rr) * NCHUNK

        i0 = 2 * pair
        one_row(0, jnp.full((L,), tau_v[pl.ds(i0, L)][0], jnp.float32))
        one_row(1, jnp.full((L,), tau_v[pl.ds(i0 + 1, L)][0], jnp.float32))

    def fire_gather(pair, gids_ref, sbuf, sem):
        pltpu.async_copy(stbl_hbm.at[gids_ref], sbuf, sem)

    def drain_gather(sbuf, sem):
        pltpu.make_async_copy(stbl_hbm.at[pl.ds(0, 2 * CAP)], sbuf, sem).wait()

    def out_win(pair):
        return out_s_hbm.at[pl.ds((base + 2 * pair) * CAP, 2 * CAP)]

    def fire_write(pair, sbuf, sem):
        pltpu.async_copy(sbuf, out_win(pair), sem)

    def drain_write(pair, sbuf, sem):
        pltpu.make_async_copy(sbuf, out_win(pair), sem).wait()

    cmbufs = ((cm_a, semc_a), (cm_b, semc_b))
    bufs = ((ids_a, gids_a, sbuf_a, semg_a, semw_a),
            (ids_b, gids_b, sbuf_b, semg_b, semw_b))
    NOCT = NPAIR // 4

    fire_cm(0, cm_a, semc_a)
    if NOCT > 1:
        fire_cm(1, cm_b, semc_b)
    for o in range(NOCT):
        cm, semc = cmbufs[o % 2]
        drain_cm(cm, semc)
        for p2 in range(4):
            g = 4 * o + p2
            ids, gids, sbuf, semg, semw = bufs[g % 2]
            scan_pair(g, cm, ids, gids)
            if g >= 2:
                drain_write(g - 2, sbuf, semw)
            fire_gather(g, gids, sbuf, semg)
            if g >= 1:
                _, _, psb, psemg, psemw = bufs[(g - 1) % 2]
                drain_gather(psb, psemg)
                fire_write(g - 1, psb, psemw)
        if o + 2 < NOCT:
            fire_cm(o + 2, cm, semc)
    _, _, lsb, lsemg, lsemw = bufs[(NPAIR - 1) % 2]
    drain_gather(lsb, lsemg)
    fire_write(NPAIR - 1, lsb, lsemw)
    drain_write(NPAIR - 2, bufs[(NPAIR - 2) % 2][2], bufs[(NPAIR - 2) % 2][4])
    drain_write(NPAIR - 1, lsb, lsemw)


@functools.lru_cache(maxsize=1)
def _k2_sc():
    return pl.kernel(
        _k2_body,
        out_type=jax.ShapeDtypeStruct((B * CAP, CW), jnp.float32),
        mesh=plsc.VectorSubcoreMesh(core_axis_name="c", subcore_axis_name="s",
                                    num_cores=NC, num_subcores=NS),
        compiler_params=pltpu.CompilerParams(needs_layout_passes=False),
        scratch_types=[
            pltpu.VMEM((8, CMOUT), jnp.float32),
            pltpu.VMEM((8, CMOUT), jnp.float32),
            pltpu.VMEM((2 * CAP,), jnp.int32),
            pltpu.VMEM((2 * CAP,), jnp.int32),
            pltpu.VMEM((2 * CAP,), jnp.int32),
            pltpu.VMEM((2 * CAP,), jnp.int32),
            pltpu.VMEM((2 * CAP, CW), jnp.float32),
            pltpu.VMEM((2 * CAP, CW), jnp.float32),
            pltpu.VMEM((ROWS_PER_W + L,), jnp.float32),
            pltpu.SemaphoreType.DMA,
            pltpu.SemaphoreType.DMA,
            pltpu.SemaphoreType.DMA,
            pltpu.SemaphoreType.DMA,
            pltpu.SemaphoreType.DMA,
            pltpu.SemaphoreType.DMA,
        ],
    )


NCAND = CAP * CW


def _k3_body(s_ref, o_ref):
    s = s_ref[...]
    lbl = lax.bitcast_convert_type(s, jnp.int32) & 15

    def it(_, c):
        lo, hi = c
        mid = (lo + hi) * 0.5
        cnt = jnp.sum((s >= mid).astype(jnp.float32), axis=1, keepdims=True)
        p = cnt >= K
        return jnp.where(p, mid, lo), jnp.where(p, hi, mid)

    lo = jnp.full((BB, 1), -1.001, jnp.float32)
    hi = jnp.full((BB, 1), 1.001, jnp.float32)
    lo, hi = lax.fori_loop(0, K3_BISECT_ITERS, it, (lo, hi))
    ms = jnp.where(s >= lo, s, 0.0)
    cols = [jnp.sum(jnp.where(lbl == c, ms, 0.0), axis=1, keepdims=True)
            for c in range(C)]
    o_ref[...] = jnp.concatenate(cols, axis=1)


def _k3(cand_s):
    return pl.pallas_call(
        _k3_body,
        grid=(B // BB,),
        in_specs=[pl.BlockSpec((BB, NCAND), lambda i: (i, 0))],
        out_specs=pl.BlockSpec((BB, C), lambda i: (i, 0)),
        out_shape=jax.ShapeDtypeStruct((B, C), jnp.float32),
    )(cand_s)


def kernel(x, keys, values):
    vals_pad = jnp.pad(values, (0, MP - M)).reshape(1, MP)
    scores, cmax, tau = _k1(x, keys, vals_pad)
    cand_s = _k2_sc()(cmax, tau.reshape(-1),
                      scores.reshape(B * NCHUNK, CW))
    return _k3(cand_s.reshape(B, NCAND))


# speedup vs baseline: 1.0731x; 1.0647x over previous
import functools

import jax
import jax.numpy as jnp
from jax import lax
from jax.experimental import pallas as pl
from jax.experimental.pallas import tpu as pltpu
from jax.experimental.pallas import tpu_sc as plsc

B = 1024
D = 512
M = 100000
K = 50
C = 10

L = 16
CW = 128
BB = 256
BM = 2048
NJ = 49
MP = NJ * BM
CPS = BM // CW
NCHUNK = MP // CW
CMOUT = 896
PAD_CHUNK = NCHUNK - 1
CAP = 64
K1_BISECT_ITERS = 22
K3_BISECT_ITERS = 38

NC = 2
NS = 16
NW = NC * NS
ROWS_PER_W = B // NW
NPAIR = ROWS_PER_W // 2


def _k1_body(x_ref, k_ref, v_ref, s_ref, cm_ref, tau_ref,
             xn_ref, cmt_ref, cmw_ref):
    j = pl.program_id(1)

    @pl.when(j == 0)
    def _():
        xv = x_ref[...]
        n = jnp.sqrt(jnp.sum(xv * xv, axis=1, keepdims=True))
        xn_ref[...] = xv / jnp.clip(n, 1e-12, None)

    s = lax.dot_general(xn_ref[...], k_ref[...], (((1,), (1,)), ((), ())),
                        preferred_element_type=jnp.float32)
    si = lax.bitcast_convert_type(s, jnp.int32)
    s = lax.bitcast_convert_type((si & ~15) | v_ref[...], jnp.float32)

    col = lax.broadcasted_iota(jnp.int32, (1, BM), 1) + j * BM
    s = jnp.where(col < M, s, -2.0)
    s3 = s.reshape(BB, CPS, CW)
    s_ref[...] = s3
    cm = jnp.max(s3, axis=2)
    cmt_ref[j] = cm

    def place(k):
        def f(old, cm16):
            parts = ([old[:, :k * CPS]] if k else []) + [cm16]
            if k < 7:
                parts.append(old[:, (k + 1) * CPS:])
            return jnp.concatenate(parts, axis=1)
        return f

    old = jnp.where(j % 8 == 0,
                    jnp.full((BB, 128), -2.0, jnp.float32), cmw_ref[...])
    cmw_ref[...] = lax.switch(j % 8, [place(k) for k in range(8)], old, cm)

    @pl.when((j % 8 == 7) | (j == NJ - 1))
    def _():
        cm_ref[...] = cmw_ref[...]

    @pl.when(j == NJ - 1)
    def _():
        cmall = cmt_ref[...]

        def it(_, c):
            lo, hi = c
            mid = (lo + hi) * 0.5
            cnt = jnp.sum(jnp.sum(
                (cmall >= mid[:, :, None]).astype(jnp.float32), axis=2),
                axis=0, keepdims=True)
            p = cnt >= K
            return jnp.where(p, mid, lo), jnp.where(p, hi, mid)

        lo = jnp.full((1, BB), -1.001, jnp.float32)
        hi = jnp.full((1, BB), 1.001, jnp.float32)
        lo, hi = lax.fori_loop(0, K1_BISECT_ITERS, it, (lo, hi))
        tau_ref[...] = lo


def _k1(x, keys, vals_pad):
    return pl.pallas_call(
        _k1_body,
        grid=(B // BB, NJ),
        in_specs=[
            pl.BlockSpec((BB, D), lambda i, j: (i, 0)),
            pl.BlockSpec((BM, D), lambda i, j: (j, 0)),
            pl.BlockSpec((1, BM), lambda i, j: (0, j)),
        ],
        out_specs=[
            pl.BlockSpec((BB, CPS, CW), lambda i, j: (i, j, 0)),
            pl.BlockSpec((BB, 128), lambda i, j: (i, j // 8)),
            pl.BlockSpec((1, BB), lambda i, j: (0, i)),
        ],
        out_shape=[
            jax.ShapeDtypeStruct((B, NCHUNK, CW), jnp.float32),
            jax.ShapeDtypeStruct((B, CMOUT), jnp.float32),
            jax.ShapeDtypeStruct((1, B), jnp.float32),
        ],
        scratch_shapes=[
            pltpu.VMEM((BB, D), jnp.float32),
            pltpu.VMEM((NJ, BB, CPS), jnp.float32),
            pltpu.VMEM((BB, 128), jnp.float32),
        ],
    )(x, keys, vals_pad)


def _k2_body(cm2_hbm, tau_hbm, stbl_hbm, out_s_hbm,
             cm_a, cm_b, ids_a, ids_b, gids_a, gids_b, sbuf_a, sbuf_b, tau_v,
             semc_a, semc_b, semg_a, semg_b, semw_a, semw_b):
    wid = lax.axis_index("s") * NC + lax.axis_index("c")
    base = wid * ROWS_PER_W
    pltpu.sync_copy(tau_hbm.at[pl.ds(base * 1, ROWS_PER_W)],
                    tau_v.at[pl.ds(0, ROWS_PER_W)])
    lane = lax.iota(jnp.int32, L)

    def cm_win(octet):
        return cm2_hbm.at[pl.ds(base + 8 * octet, 8), :]

    def fire_cm(octet, cm_ref, sem):
        return pltpu.async_copy(cm_win(octet), cm_ref, sem)

    def drain_cm(cm_ref, sem):
        pltpu.make_async_copy(cm_win(0), cm_ref, sem).wait()

    def scan_pair(pair, cm_ref, ids_ref, gids_ref):
        r0 = base + 2 * pair
        p2 = pair % 4

        def one_row(rr, tau_vec):
            off = CAP * rr

            def body(jj, pos):
                v = cm_ref[2 * p2 + rr, pl.ds(jj * L, L)]
                msk = v >= tau_vec
                ids = lane + jj * L
                mi = jnp.where(msk, 1, 0)
                slots = off + pos + plsc.cumsum(mi) - 1
                plsc.store_scatter(ids_ref, [slots], ids,
                                   mask=msk & (slots < off + CAP))
                return jnp.minimum(pos + jnp.sum(mi), CAP)

            npos = lax.fori_loop(0, NCHUNK // L, body, jnp.int32(0))
            for k in range(CAP // L):
                sl = ids_ref[pl.ds(off + k * L, L)]
                slot = lane + k * L
                sl = jnp.where(slot >= npos, PAD_CHUNK, sl)
                ids_ref[pl.ds(off + k * L, L)] = sl
                gids_ref[pl.ds(off + k * L, L)] = sl + (r0 + rr) * NCHUNK

        i0 = 2 * pair
        one_row(0, jnp.full((L,), tau_v[pl.ds(i0, L)][0], jnp.float32))
        one_row(1, jnp.full((L,), tau_v[pl.ds(i0 + 1, L)][0], jnp.float32))

    def fire_gather(pair, gids_ref, sbuf, sem):
        pltpu.async_copy(stbl_hbm.at[gids_ref], sbuf, sem)

    def drain_gather(sbuf, sem):
        pltpu.make_async_copy(stbl_hbm.at[pl.ds(0, 2 * CAP)], sbuf, sem).wait()

    def out_win(pair):
        return out_s_hbm.at[pl.ds((base + 2 * pair) * CAP, 2 * CAP)]

    def fire_write(pair, sbuf, sem):
        pltpu.async_copy(sbuf, out_win(pair), sem)

    def drain_write(pair, sbuf, sem):
        pltpu.make_async_copy(sbuf, out_win(pair), sem).wait()

    cmbufs = ((cm_a, semc_a), (cm_b, semc_b))
    bufs = ((ids_a, gids_a, sbuf_a, semg_a, semw_a),
            (ids_b, gids_b, sbuf_b, semg_b, semw_b))
    NOCT = NPAIR // 4

    fire_cm(0, cm_a, semc_a)
    if NOCT > 1:
        fire_cm(1, cm_b, semc_b)
    for o in range(NOCT):
        cm, semc = cmbufs[o % 2]
        drain_cm(cm, semc)
        for p2 in range(4):
            g = 4 * o + p2
            ids, gids, sbuf, semg, semw = bufs[g % 2]
            scan_pair(g, cm, ids, gids)
            if g >= 2:
                drain_write(g - 2, sbuf, semw)
            fire_gather(g, gids, sbuf, semg)
            if g >= 1:
                _, _, psb, psemg, psemw = bufs[(g - 1) % 2]
                drain_gather(psb, psemg)
                fire_write(g - 1, psb, psemw)
        if o + 2 < NOCT:
            fire_cm(o + 2, cm, semc)
    _, _, lsb, lsemg, lsemw = bufs[(NPAIR - 1) % 2]
    drain_gather(lsb, lsemg)
    fire_write(NPAIR - 1, lsb, lsemw)
    drain_write(NPAIR - 2, bufs[(NPAIR - 2) % 2][2], bufs[(NPAIR - 2) % 2][4])
    drain_write(NPAIR - 1, lsb, lsemw)


@functools.lru_cache(maxsize=1)
def _k2_sc():
    return pl.kernel(
        _k2_body,
        out_type=jax.ShapeDtypeStruct((B * CAP, CW), jnp.float32),
        mesh=plsc.VectorSubcoreMesh(core_axis_name="c", subcore_axis_name="s",
                                    num_cores=NC, num_subcores=NS),
        compiler_params=pltpu.CompilerParams(needs_layout_passes=False),
        scratch_types=[
            pltpu.VMEM((8, CMOUT), jnp.float32),
            pltpu.VMEM((8, CMOUT), jnp.float32),
            pltpu.VMEM((2 * CAP,), jnp.int32),
            pltpu.VMEM((2 * CAP,), jnp.int32),
            pltpu.VMEM((2 * CAP,), jnp.int32),
            pltpu.VMEM((2 * CAP,), jnp.int32),
            pltpu.VMEM((2 * CAP, CW), jnp.float32),
            pltpu.VMEM((2 * CAP, CW), jnp.float32),
            pltpu.VMEM((ROWS_PER_W + L,), jnp.float32),
            pltpu.SemaphoreType.DMA,
            pltpu.SemaphoreType.DMA,
            pltpu.SemaphoreType.DMA,
            pltpu.SemaphoreType.DMA,
            pltpu.SemaphoreType.DMA,
            pltpu.SemaphoreType.DMA,
        ],
    )


NCAND = CAP * CW


def _k3_body(s_ref, o_ref):
    s = s_ref[...]
    lbl = lax.bitcast_convert_type(s, jnp.int32) & 15

    def it(_, c):
        lo, hi = c
        mid = (lo + hi) * 0.5
        cnt = jnp.sum((s >= mid).astype(jnp.float32), axis=1, keepdims=True)
        p = cnt >= K
        return jnp.where(p, mid, lo), jnp.where(p, hi, mid)

    lo = jnp.full((BB, 1), -1.001, jnp.float32)
    hi = jnp.full((BB, 1), 1.001, jnp.float32)
    lo, hi = lax.fori_loop(0, K3_BISECT_ITERS, it, (lo, hi))
    ms = jnp.where(s >= lo, s, 0.0)
    cols = [jnp.sum(jnp.where(lbl == c, ms, 0.0), axis=1, keepdims=True)
            for c in range(C)]
    o_ref[...] = jnp.concatenate(cols, axis=1)


def _k3(cand_s):
    return pl.pallas_call(
        _k3_body,
        grid=(B // BB,),
        in_specs=[pl.BlockSpec((BB, NCAND), lambda i: (i, 0))],
        out_specs=pl.BlockSpec((BB, C), lambda i: (i, 0)),
        out_shape=jax.ShapeDtypeStruct((B, C), jnp.float32),
    )(cand_s)


def kernel(x, keys, values):
    vals_pad = jnp.pad(values, (0, MP - M)).reshape(1, MP)
    scores, cmax, tau = _k1(x, keys, vals_pad)
    cand_s = _k2_sc()(cmax, tau.reshape(-1),
                      scores.reshape(B * NCHUNK, CW))
    return _k3(cand_s.reshape(B, NCAND))
